# baseline (speedup 1.0000x reference)
import functools

import jax
import jax.numpy as jnp
from jax import lax
from jax.experimental import pallas as pl
from jax.experimental.pallas import tpu as pltpu
from jax.experimental.pallas import tpu_sc as plsc

DM = 768
NLAYER = 12
NEXP = 4
FF = 3072
PS = 16
NPATCH = 196
NHEAD = 12
DH = 64
NCLS = 1000
BS = 4
SEQ = 197
SEQP = 200
NTOK = BS * SEQP
TILE = 64
NTILE = 16
NSORT = NTILE * TILE
EPAD = 128
NEG = -1e30


def _dispatch_mat(meta, rows):
    dest_i = meta[:, 0:1].astype(jnp.int32)
    gate = meta[:, 1:2]
    slot_i = lax.broadcasted_iota(jnp.int32, (rows, NSORT), 1)
    return jnp.where(slot_i == dest_i, gate, 0.0)


def _layer_body(li_ref, h_ref, yprev_ref, mprev_ref, l1s_ref, l1b_ref,
                wqkv_ref, bqkv_ref, wo_ref, bo_ref, l2s_ref, l2b_ref, gw_ref,
                hout_ref, z_ref, perm_ref, meta_ref, etile_ref):
    b = pl.program_id(0)
    dmat = _dispatch_mat(mprev_ref[...], SEQP)
    hb = h_ref[0] + jnp.dot(dmat, yprev_ref[...],
                            preferred_element_type=jnp.float32)

    m = jnp.mean(hb, axis=-1, keepdims=True)
    va = jnp.mean((hb - m) ** 2, axis=-1, keepdims=True)
    qin = (hb - m) * lax.rsqrt(va + 1e-5) * l1s_ref[0] + l1b_ref[0]
    wqkv = wqkv_ref[0]
    bqkv_ = bqkv_ref[0]
    q = jnp.dot(qin, wqkv[:, :DM],
                preferred_element_type=jnp.float32) + bqkv_[:, :DM]
    k = jnp.dot(hb, wqkv[:, DM:2 * DM],
                preferred_element_type=jnp.float32) + bqkv_[:, DM:2 * DM]
    v = jnp.dot(hb, wqkv[:, 2 * DM:],
                preferred_element_type=jnp.float32) + bqkv_[:, 2 * DM:]
    colpad = lax.broadcasted_iota(jnp.int32, (SEQP, SEQP), 1) >= SEQ
    kmask = jnp.where(colpad, NEG, 0.0)
    parts = []
    for hh in range(NHEAD):
        qh = q[:, hh * DH:(hh + 1) * DH]
        kh = k[:, hh * DH:(hh + 1) * DH]
        vh = v[:, hh * DH:(hh + 1) * DH]
        lg = lax.dot_general(qh, kh, (((1,), (1,)), ((), ())),
                             preferred_element_type=jnp.float32) * 0.125
        lg = lg + kmask
        mx = jnp.max(lg, axis=-1, keepdims=True)
        ex = jnp.exp(lg - mx)
        att = ex / jnp.sum(ex, axis=-1, keepdims=True)
        parts.append(jnp.dot(att, vh, preferred_element_type=jnp.float32))
    o = jnp.concatenate(parts, axis=1)
    o = jnp.dot(o, wo_ref[0], preferred_element_type=jnp.float32) + bo_ref[0]
    rowpad = lax.broadcasted_iota(jnp.int32, (SEQP, DM), 0) >= SEQ
    hn = hb + jnp.where(rowpad, 0.0, o)
    hout_ref[0] = hn

    m2 = jnp.mean(hn, axis=-1, keepdims=True)
    v2 = jnp.mean((hn - m2) ** 2, axis=-1, keepdims=True)
    z_b = (hn - m2) * lax.rsqrt(v2 + 1e-5) * l2s_ref[0] + l2b_ref[0]
    z_ref[pl.ds(b * SEQP, SEQP), :] = z_b

    @pl.when(b == BS - 1)
    def _():
        z = z_ref[...]
        lane_i = lax.broadcasted_iota(jnp.int32, (NTOK, EPAD), 1)
        bias = jnp.where(lane_i < NEXP, 0.0, NEG)
        logits = jnp.dot(z, gw_ref[0], preferred_element_type=jnp.float32) + bias
        mx = jnp.max(logits, axis=1, keepdims=True)
        ex = jnp.exp(logits - mx)
        den = jnp.sum(ex, axis=1, keepdims=True)
        gate = jnp.max(ex, axis=1, keepdims=True) / den
        idx_i = jnp.min(jnp.where(logits >= mx, lane_i, EPAD), axis=1,
                        keepdims=True)

        trow_i = lax.broadcasted_iota(jnp.int32, (NTOK, 1), 0)
        is_real = (trow_i % SEQP) < SEQ
        onehot = jnp.where((lane_i == idx_i) & is_real, 1.0, 0.0)

        rr = lax.broadcasted_iota(jnp.int32, (NTOK, NTOK), 0)
        cc = lax.broadcasted_iota(jnp.int32, (NTOK, NTOK), 1)
        tril = jnp.where(rr > cc, 1.0, 0.0)
        cum = jnp.dot(tril, onehot, preferred_element_type=jnp.float32)
        rank = jnp.sum(cum * onehot, axis=1, keepdims=True)
        counts = jnp.sum(onehot, axis=0, keepdims=True)
        pc = jnp.floor((counts + float(TILE - 1)) / float(TILE)) * float(TILE)
        r2 = lax.broadcasted_iota(jnp.int32, (EPAD, EPAD), 0)
        c2 = lax.broadcasted_iota(jnp.int32, (EPAD, EPAD), 1)
        offs = jnp.dot(pc, jnp.where(r2 < c2, 1.0, 0.0),
                       preferred_element_type=jnp.float32)
        ends = offs + pc
        off_t = jnp.sum(onehot * offs, axis=1, keepdims=True)
        dest = off_t + rank
        dest_i = dest.astype(jnp.int32)
        gate = jnp.where(is_real, gate, 0.0)
        meta_ref[...] = (jnp.where(lane_i == 0, dest, 0.0) +
                         jnp.where(lane_i == 1, gate, 0.0))

        slot_i = lax.broadcasted_iota(jnp.int32, (NTOK, NSORT), 1)
        pmat = jnp.where((slot_i == dest_i) & is_real, 1.0, 0.0)
        trowf = trow_i.astype(jnp.float32)
        permf = jnp.sum(pmat * trowf, axis=0, keepdims=True)
        occ = jnp.sum(pmat, axis=0, keepdims=True)
        permr = jnp.where(occ > 0.5, permf, float(NTOK - 1))
        perm_ref[...] = jnp.broadcast_to(permr, (8, NSORT)).astype(jnp.int32)

        gstart = lax.broadcasted_iota(jnp.int32, (8, EPAD), 1).astype(
            jnp.float32) * float(TILE)
        et = jnp.zeros((8, EPAD), jnp.float32)
        for e in range(NEXP):
            end_e = lax.slice(ends, (0, e), (1, e + 1))
            et = et + jnp.where(gstart >= end_e, 1.0, 0.0)
        etile_ref[...] = jnp.clip(et, 0.0, float(NEXP - 1)).astype(jnp.int32)


def _layer_call(li, h, yprev, mprev, l1s3, l1b3, wqkv, bqkv3, wo, bo3, l2s3,
                l2b3, gwp):
    grid_spec = pltpu.PrefetchScalarGridSpec(
        num_scalar_prefetch=1,
        grid=(BS,),
        in_specs=[
            pl.BlockSpec((1, SEQP, DM), lambda b, li_r: (b, 0, 0)),
            pl.BlockSpec((NSORT, DM), lambda b, li_r: (0, 0)),
            pl.BlockSpec((SEQP, EPAD), lambda b, li_r: (b, 0)),
            pl.BlockSpec((1, 1, DM), lambda b, li_r: (li_r[0], 0, 0)),
            pl.BlockSpec((1, 1, DM), lambda b, li_r: (li_r[0], 0, 0)),
            pl.BlockSpec((1, DM, 3 * DM), lambda b, li_r: (li_r[0], 0, 0)),
            pl.BlockSpec((1, 1, 3 * DM), lambda b, li_r: (li_r[0], 0, 0)),
            pl.BlockSpec((1, DM, DM), lambda b, li_r: (li_r[0], 0, 0)),
            pl.BlockSpec((1, 1, DM), lambda b, li_r: (li_r[0], 0, 0)),
            pl.BlockSpec((1, 1, DM), lambda b, li_r: (li_r[0], 0, 0)),
            pl.BlockSpec((1, 1, DM), lambda b, li_r: (li_r[0], 0, 0)),
            pl.BlockSpec((1, DM, EPAD), lambda b, li_r: (li_r[0], 0, 0)),
        ],
        out_specs=[
            pl.BlockSpec((1, SEQP, DM), lambda b, li_r: (b, 0, 0)),
            pl.BlockSpec((NTOK, DM), lambda b, li_r: (0, 0)),
            pl.BlockSpec((8, NSORT), lambda b, li_r: (0, 0)),
            pl.BlockSpec((NTOK, EPAD), lambda b, li_r: (0, 0)),
            pl.BlockSpec((8, EPAD), lambda b, li_r: (0, 0)),
        ],
    )
    return pl.pallas_call(
        _layer_body,
        grid_spec=grid_spec,
        out_shape=(
            jax.ShapeDtypeStruct((BS, SEQP, DM), jnp.float32),
            jax.ShapeDtypeStruct((NTOK, DM), jnp.float32),
            jax.ShapeDtypeStruct((8, NSORT), jnp.int32),
            jax.ShapeDtypeStruct((NTOK, EPAD), jnp.float32),
            jax.ShapeDtypeStruct((8, EPAD), jnp.int32),
        ),
    )(li, h, yprev, mprev, l1s3, l1b3, wqkv, bqkv3, wo, bo3, l2s3, l2b3, gwp)


SC_CHUNK = NSORT // 32


@functools.lru_cache(maxsize=1)
def _sc_gather_kernel():
    @functools.partial(
        pl.kernel,
        out_type=jax.ShapeDtypeStruct((NSORT, DM), jnp.float32),
        mesh=plsc.VectorSubcoreMesh(core_axis_name="c", subcore_axis_name="s"),
        scratch_types=[
            pltpu.VMEM((SC_CHUNK,), jnp.int32),
            pltpu.VMEM((SC_CHUNK, DM), jnp.float32),
            pltpu.SemaphoreType.DMA,
        ],
    )
    def _sc_gather(z_hbm, idx_hbm, out_hbm, idx_v, rows_v, sem):
        wid = lax.axis_index("s") * 2 + lax.axis_index("c")
        base = wid * SC_CHUNK
        pltpu.sync_copy(idx_hbm.at[pl.ds(base, SC_CHUNK)], idx_v)
        pltpu.async_copy(z_hbm.at[idx_v], rows_v, sem).wait()
        pltpu.sync_copy(rows_v, out_hbm.at[pl.ds(base, SC_CHUNK)])

    return _sc_gather


def _ffn_body(li_ref, emap_ref, z_ref, w1_ref, b1_ref, w2_ref, b2_ref,
              out_ref):
    zt = z_ref[...]
    h1 = jnp.dot(zt, w1_ref[0, 0],
                 preferred_element_type=jnp.float32) + b1_ref[0, 0]
    a = jax.nn.gelu(h1)
    out_ref[...] = jnp.dot(a, w2_ref[0, 0],
                           preferred_element_type=jnp.float32) + b2_ref[0, 0]


def _ffn_call(li, emap, z_sorted, w1, b1r, w2, b2r):
    grid_spec = pltpu.PrefetchScalarGridSpec(
        num_scalar_prefetch=2,
        grid=(NTILE,),
        in_specs=[
            pl.BlockSpec((TILE, DM), lambda g, li_r, em: (g, 0)),
            pl.BlockSpec((1, 1, DM, FF),
                         lambda g, li_r, em: (li_r[0], em[g], 0, 0)),
            pl.BlockSpec((1, 1, 1, FF),
                         lambda g, li_r, em: (li_r[0], em[g], 0, 0)),
            pl.BlockSpec((1, 1, FF, DM),
                         lambda g, li_r, em: (li_r[0], em[g], 0, 0)),
            pl.BlockSpec((1, 1, 1, DM),
                         lambda g, li_r, em: (li_r[0], em[g], 0, 0)),
        ],
        out_specs=pl.BlockSpec((TILE, DM), lambda g, li_r, em: (g, 0)),
    )
    return pl.pallas_call(
        _ffn_body,
        grid_spec=grid_spec,
        out_shape=jax.ShapeDtypeStruct((NSORT, DM), jnp.float32),
    )(li, emap, z_sorted, w1, b1r, w2, b2r)


def _patch_body(p_ref, w_ref, b_ref, pos_ref, out_ref):
    out_ref[...] = (jnp.dot(p_ref[...], w_ref[...],
                            preferred_element_type=jnp.float32)
                    + b_ref[...] + pos_ref[...])


def _patch_call(p, w, b, pos_t):
    return pl.pallas_call(
        _patch_body,
        out_shape=jax.ShapeDtypeStruct((BS * NPATCH, DM), jnp.float32),
    )(p, w, b, pos_t)


def _head_body(h_ref, m_ref, y_ref, w_ref, b_ref, out_ref):
    dmat = _dispatch_mat(m_ref[...], 8)
    hc = h_ref[...] + jnp.dot(dmat, y_ref[...],
                              preferred_element_type=jnp.float32)
    out_ref[...] = jnp.dot(hc, w_ref[...],
                           preferred_element_type=jnp.float32) + b_ref[...]


def _head_call(hcls8, mcls8, y, w, b):
    return pl.pallas_call(
        _head_body,
        out_shape=jax.ShapeDtypeStruct((8, 1024), jnp.float32),
    )(hcls8, mcls8, y, w, b)


def kernel(x, W_patch, b_patch, cls_token, pos_embed, ln1_s, ln1_b, Wqkv,
           bqkv, Wo, bo, ln2_s, ln2_b, gate_W, W1, b1, W2, b2, W_head,
           b_head):
    p = (x.reshape(BS, 3, 224 // PS, PS, 224 // PS, PS)
         .transpose(0, 2, 4, 1, 3, 5).reshape(BS * NPATCH, 3 * PS * PS))
    pos_t = jnp.tile(pos_embed[0, 1:, :], (BS, 1))
    t = _patch_call(p, W_patch, b_patch.reshape(1, DM), pos_t)
    cls = cls_token[0] + pos_embed[0, 0:1, :]
    h = jnp.concatenate(
        [jnp.broadcast_to(cls[None], (BS, 1, DM)), t.reshape(BS, NPATCH, DM)],
        axis=1)
    h = jnp.pad(h, ((0, 0), (0, SEQP - SEQ), (0, 0)))

    gwp = jnp.pad(gate_W, ((0, 0), (0, 0), (0, EPAD - NEXP)))
    l1s3 = ln1_s.reshape(NLAYER, 1, DM)
    l1b3 = ln1_b.reshape(NLAYER, 1, DM)
    bqkv3 = bqkv.reshape(NLAYER, 1, 3 * DM)
    bo3 = bo.reshape(NLAYER, 1, DM)
    l2s3 = ln2_s.reshape(NLAYER, 1, DM)
    l2b3 = ln2_b.reshape(NLAYER, 1, DM)
    b1r = b1.reshape(NLAYER, NEXP, 1, FF)
    b2r = b2.reshape(NLAYER, NEXP, 1, DM)

    y = jnp.zeros((NSORT, DM), jnp.float32)
    mt = jnp.zeros((NTOK, EPAD), jnp.float32)
    for i in range(NLAYER):
        li = jnp.full((1,), i, jnp.int32)
        h, z, perm8, mt, etile8 = _layer_call(
            li, h, y, mt, l1s3, l1b3, Wqkv, bqkv3, Wo, bo3, l2s3, l2b3, gwp)
        z_sorted = _sc_gather_kernel()(z, perm8[0])
        y = _ffn_call(li, etile8[0, :NTILE], z_sorted, W1, b1r, W2, b2r)

    hcls = h.reshape(NTOK, DM)[::SEQP]
    hcls8 = jnp.pad(hcls, ((0, 8 - BS), (0, 0)))
    mcls8 = jnp.pad(mt[::SEQP], ((0, 8 - BS), (0, 0)))
    whp = jnp.pad(W_head, ((0, 0), (0, 1024 - NCLS)))
    bhp = jnp.pad(b_head, (0, 1024 - NCLS)).reshape(1, 1024)
    out = _head_call(hcls8, mcls8, y, whp, bhp)
    return out[:BS, :NCLS]


# speedup vs baseline: 1.0927x; 1.0927x over previous
import functools

import jax
import jax.numpy as jnp
from jax import lax
from jax.experimental import pallas as pl
from jax.experimental.pallas import tpu as pltpu
from jax.experimental.pallas import tpu_sc as plsc

DM = 768
NLAYER = 12
NEXP = 4
FF = 3072
PS = 16
NPATCH = 196
NHEAD = 12
DH = 64
NCLS = 1000
BS = 4
SEQ = 197
SEQP = 200
NTOK = BS * SEQP
TILE = 64
NTILE = 16
NSORT = NTILE * TILE
EPAD = 128
NEG = -1e30


def _dispatch_mat(meta, rows):
    dest_i = meta[:, 0:1].astype(jnp.int32)
    gate = meta[:, 1:2]
    slot_i = lax.broadcasted_iota(jnp.int32, (rows, NSORT), 1)
    return jnp.where(slot_i == dest_i, gate, 0.0)


def _layer_body(li_ref, h_ref, yprev_ref, mprev_ref, l1s_ref, l1b_ref,
                wqkv_ref, bqkv_ref, wo_ref, bo_ref, l2s_ref, l2b_ref, gw_ref,
                hout_ref, z_ref, perm_ref, meta_ref, estart_ref, ecnt_ref):
    b = pl.program_id(0)
    dmat = _dispatch_mat(mprev_ref[...], SEQP)
    hb = h_ref[0] + jnp.dot(dmat, yprev_ref[...],
                            preferred_element_type=jnp.float32)

    m = jnp.mean(hb, axis=-1, keepdims=True)
    va = jnp.mean((hb - m) ** 2, axis=-1, keepdims=True)
    qin = (hb - m) * lax.rsqrt(va + 1e-5) * l1s_ref[0] + l1b_ref[0]
    wqkv = wqkv_ref[0]
    bqkv_ = bqkv_ref[0]
    q = jnp.dot(qin, wqkv[:, :DM],
                preferred_element_type=jnp.float32) + bqkv_[:, :DM]
    k = jnp.dot(hb, wqkv[:, DM:2 * DM],
                preferred_element_type=jnp.float32) + bqkv_[:, DM:2 * DM]
    v = jnp.dot(hb, wqkv[:, 2 * DM:],
                preferred_element_type=jnp.float32) + bqkv_[:, 2 * DM:]
    colpad = lax.broadcasted_iota(jnp.int32, (SEQP, SEQP), 1) >= SEQ
    kmask = jnp.where(colpad, NEG, 0.0)
    parts = []
    for hh in range(NHEAD):
        qh = q[:, hh * DH:(hh + 1) * DH]
        kh = k[:, hh * DH:(hh + 1) * DH]
        vh = v[:, hh * DH:(hh + 1) * DH]
        lg = lax.dot_general(qh, kh, (((1,), (1,)), ((), ())),
                             preferred_element_type=jnp.float32) * 0.125
        lg = lg + kmask
        mx = jnp.max(lg, axis=-1, keepdims=True)
        ex = jnp.exp(lg - mx)
        att = ex / jnp.sum(ex, axis=-1, keepdims=True)
        parts.append(jnp.dot(att, vh, preferred_element_type=jnp.float32))
    o = jnp.concatenate(parts, axis=1)
    o = jnp.dot(o, wo_ref[0], preferred_element_type=jnp.float32) + bo_ref[0]
    rowpad = lax.broadcasted_iota(jnp.int32, (SEQP, DM), 0) >= SEQ
    hn = hb + jnp.where(rowpad, 0.0, o)
    hout_ref[0] = hn

    m2 = jnp.mean(hn, axis=-1, keepdims=True)
    v2 = jnp.mean((hn - m2) ** 2, axis=-1, keepdims=True)
    z_b = (hn - m2) * lax.rsqrt(v2 + 1e-5) * l2s_ref[0] + l2b_ref[0]
    z_ref[pl.ds(b * SEQP, SEQP), :] = z_b

    @pl.when(b == BS - 1)
    def _():
        z = z_ref[...]
        lane_i = lax.broadcasted_iota(jnp.int32, (NTOK, EPAD), 1)
        bias = jnp.where(lane_i < NEXP, 0.0, NEG)
        logits = jnp.dot(z, gw_ref[0], preferred_element_type=jnp.float32) + bias
        mx = jnp.max(logits, axis=1, keepdims=True)
        ex = jnp.exp(logits - mx)
        den = jnp.sum(ex, axis=1, keepdims=True)
        gate = jnp.max(ex, axis=1, keepdims=True) / den
        idx_i = jnp.min(jnp.where(logits >= mx, lane_i, EPAD), axis=1,
                        keepdims=True)

        trow_i = lax.broadcasted_iota(jnp.int32, (NTOK, 1), 0)
        is_real = (trow_i % SEQP) < SEQ
        onehot = jnp.where((lane_i == idx_i) & is_real, 1.0, 0.0)

        rr = lax.broadcasted_iota(jnp.int32, (NTOK, NTOK), 0)
        cc = lax.broadcasted_iota(jnp.int32, (NTOK, NTOK), 1)
        tril = jnp.where(rr > cc, 1.0, 0.0)
        cum = jnp.dot(tril, onehot, preferred_element_type=jnp.float32)
        rank = jnp.sum(cum * onehot, axis=1, keepdims=True)
        counts = jnp.sum(onehot, axis=0, keepdims=True)
        pc = jnp.floor((counts + float(TILE - 1)) / float(TILE)) * float(TILE)
        r2 = lax.broadcasted_iota(jnp.int32, (EPAD, EPAD), 0)
        c2 = lax.broadcasted_iota(jnp.int32, (EPAD, EPAD), 1)
        offs = jnp.dot(pc, jnp.where(r2 < c2, 1.0, 0.0),
                       preferred_element_type=jnp.float32)
        ends = offs + pc
        off_t = jnp.sum(onehot * offs, axis=1, keepdims=True)
        dest = off_t + rank
        dest_i = dest.astype(jnp.int32)
        gate = jnp.where(is_real, gate, 0.0)
        meta_ref[...] = (jnp.where(lane_i == 0, dest, 0.0) +
                         jnp.where(lane_i == 1, gate, 0.0))

        slot_i = lax.broadcasted_iota(jnp.int32, (NTOK, NSORT), 1)
        pmat = jnp.where((slot_i == dest_i) & is_real, 1.0, 0.0)
        trowf = trow_i.astype(jnp.float32)
        permf = jnp.sum(pmat * trowf, axis=0, keepdims=True)
        occ = jnp.sum(pmat, axis=0, keepdims=True)
        permr = jnp.where(occ > 0.5, permf, float(NTOK - 1))
        perm_ref[...] = jnp.broadcast_to(permr, (8, NSORT)).astype(jnp.int32)

        estart_ref[...] = jnp.broadcast_to(
            offs / float(TILE), (8, EPAD)).astype(jnp.int32)
        ecnt_ref[...] = jnp.broadcast_to(
            pc / float(TILE), (8, EPAD)).astype(jnp.int32)


def _layer_call(li, h, yprev, mprev, l1s3, l1b3, wqkv, bqkv3, wo, bo3, l2s3,
                l2b3, gwp):
    grid_spec = pltpu.PrefetchScalarGridSpec(
        num_scalar_prefetch=1,
        grid=(BS,),
        in_specs=[
            pl.BlockSpec((1, SEQP, DM), lambda b, li_r: (b, 0, 0)),
            pl.BlockSpec((NSORT, DM), lambda b, li_r: (0, 0)),
            pl.BlockSpec((SEQP, EPAD), lambda b, li_r: (b, 0)),
            pl.BlockSpec((1, 1, DM), lambda b, li_r: (li_r[0], 0, 0)),
            pl.BlockSpec((1, 1, DM), lambda b, li_r: (li_r[0], 0, 0)),
            pl.BlockSpec((1, DM, 3 * DM), lambda b, li_r: (li_r[0], 0, 0)),
            pl.BlockSpec((1, 1, 3 * DM), lambda b, li_r: (li_r[0], 0, 0)),
            pl.BlockSpec((1, DM, DM), lambda b, li_r: (li_r[0], 0, 0)),
            pl.BlockSpec((1, 1, DM), lambda b, li_r: (li_r[0], 0, 0)),
            pl.BlockSpec((1, 1, DM), lambda b, li_r: (li_r[0], 0, 0)),
            pl.BlockSpec((1, 1, DM), lambda b, li_r: (li_r[0], 0, 0)),
            pl.BlockSpec((1, DM, EPAD), lambda b, li_r: (li_r[0], 0, 0)),
        ],
        out_specs=[
            pl.BlockSpec((1, SEQP, DM), lambda b, li_r: (b, 0, 0)),
            pl.BlockSpec((NTOK, DM), lambda b, li_r: (0, 0)),
            pl.BlockSpec((8, NSORT), lambda b, li_r: (0, 0)),
            pl.BlockSpec((NTOK, EPAD), lambda b, li_r: (0, 0)),
            pl.BlockSpec((8, EPAD), lambda b, li_r: (0, 0)),
            pl.BlockSpec((8, EPAD), lambda b, li_r: (0, 0)),
        ],
    )
    return pl.pallas_call(
        _layer_body,
        grid_spec=grid_spec,
        out_shape=(
            jax.ShapeDtypeStruct((BS, SEQP, DM), jnp.float32),
            jax.ShapeDtypeStruct((NTOK, DM), jnp.float32),
            jax.ShapeDtypeStruct((8, NSORT), jnp.int32),
            jax.ShapeDtypeStruct((NTOK, EPAD), jnp.float32),
            jax.ShapeDtypeStruct((8, EPAD), jnp.int32),
            jax.ShapeDtypeStruct((8, EPAD), jnp.int32),
        ),
    )(li, h, yprev, mprev, l1s3, l1b3, wqkv, bqkv3, wo, bo3, l2s3, l2b3, gwp)


SC_CHUNK = NSORT // 32


@functools.lru_cache(maxsize=1)
def _sc_gather_kernel():
    @functools.partial(
        pl.kernel,
        out_type=jax.ShapeDtypeStruct((NSORT, DM), jnp.float32),
        mesh=plsc.VectorSubcoreMesh(core_axis_name="c", subcore_axis_name="s"),
        scratch_types=[
            pltpu.VMEM((SC_CHUNK,), jnp.int32),
            pltpu.VMEM((SC_CHUNK, DM), jnp.float32),
            pltpu.SemaphoreType.DMA,
        ],
    )
    def _sc_gather(z_hbm, idx_hbm, out_hbm, idx_v, rows_v, sem):
        wid = lax.axis_index("s") * 2 + lax.axis_index("c")
        base = wid * SC_CHUNK
        pltpu.sync_copy(idx_hbm.at[pl.ds(base, SC_CHUNK)], idx_v)
        pltpu.async_copy(z_hbm.at[idx_v], rows_v, sem).wait()
        pltpu.sync_copy(rows_v, out_hbm.at[pl.ds(base, SC_CHUNK)])

    return _sc_gather


def _ffn_body(li_ref, es_ref, ec_ref, z_ref, w1_ref, b1_ref, w2_ref, b2_ref,
              out_ref, w1a, w1b_, w2a, w2b_, s1a, s1b, s2a, s2b):
    li = li_ref[0]
    w1bufs, w2bufs = (w1a, w1b_), (w2a, w2b_)
    s1, s2 = (s1a, s1b), (s2a, s2b)
    out_ref[...] = jnp.zeros((NSORT, DM), jnp.float32)

    def fetch(e):
        pltpu.make_async_copy(w1_ref.at[li, e], w1bufs[e % 2], s1[e % 2]).start()
        pltpu.make_async_copy(w2_ref.at[li, e], w2bufs[e % 2], s2[e % 2]).start()

    fetch(0)
    fetch(1)
    for e in range(NEXP):
        pltpu.make_async_copy(w1_ref.at[li, e], w1bufs[e % 2], s1[e % 2]).wait()
        pltpu.make_async_copy(w2_ref.at[li, e], w2bufs[e % 2], s2[e % 2]).wait()
        w1e = w1bufs[e % 2][...]
        w2e = w2bufs[e % 2][...]
        b1e = b1_ref[0, e]
        b2e = b2_ref[0, e]
        start = es_ref[e]
        cnt = ec_ref[e]

        def tile_step(i, _):
            r = pl.multiple_of((start + i) * TILE, TILE)
            zt = z_ref[pl.ds(r, TILE), :]
            h1 = jnp.dot(zt, w1e, preferred_element_type=jnp.float32) + b1e
            a = jax.nn.gelu(h1)
            out_ref[pl.ds(r, TILE), :] = jnp.dot(
                a, w2e, preferred_element_type=jnp.float32) + b2e
            return 0

        lax.fori_loop(0, cnt, tile_step, 0)
        if e + 2 < NEXP:
            fetch(e + 2)


def _ffn_call(li, es, ec, z_sorted, w1, b1r, w2, b2r):
    grid_spec = pltpu.PrefetchScalarGridSpec(
        num_scalar_prefetch=3,
        grid=(1,),
        in_specs=[
            pl.BlockSpec((NSORT, DM), lambda g, li_r, es_r, ec_r: (0, 0)),
            pl.BlockSpec(memory_space=pltpu.MemorySpace.HBM),
            pl.BlockSpec((1, NEXP, 1, FF),
                         lambda g, li_r, es_r, ec_r: (li_r[0], 0, 0, 0)),
            pl.BlockSpec(memory_space=pltpu.MemorySpace.HBM),
            pl.BlockSpec((1, NEXP, 1, DM),
                         lambda g, li_r, es_r, ec_r: (li_r[0], 0, 0, 0)),
        ],
        out_specs=pl.BlockSpec((NSORT, DM), lambda g, li_r, es_r, ec_r: (0, 0)),
        scratch_shapes=[
            pltpu.VMEM((DM, FF), jnp.float32),
            pltpu.VMEM((DM, FF), jnp.float32),
            pltpu.VMEM((FF, DM), jnp.float32),
            pltpu.VMEM((FF, DM), jnp.float32),
            pltpu.SemaphoreType.DMA,
            pltpu.SemaphoreType.DMA,
            pltpu.SemaphoreType.DMA,
            pltpu.SemaphoreType.DMA,
        ],
    )
    return pl.pallas_call(
        _ffn_body,
        grid_spec=grid_spec,
        out_shape=jax.ShapeDtypeStruct((NSORT, DM), jnp.float32),
        compiler_params=pltpu.CompilerParams(
            vmem_limit_bytes=100 * 1024 * 1024),
    )(li, es, ec, z_sorted, w1, b1r, w2, b2r)


def _patch_body(p_ref, w_ref, b_ref, pos_ref, out_ref):
    out_ref[...] = (jnp.dot(p_ref[...], w_ref[...],
                            preferred_element_type=jnp.float32)
                    + b_ref[...] + pos_ref[...])


def _patch_call(p, w, b, pos_t):
    return pl.pallas_call(
        _patch_body,
        out_shape=jax.ShapeDtypeStruct((BS * NPATCH, DM), jnp.float32),
    )(p, w, b, pos_t)


def _head_body(h_ref, m_ref, y_ref, w_ref, b_ref, out_ref):
    dmat = _dispatch_mat(m_ref[...], 8)
    hc = h_ref[...] + jnp.dot(dmat, y_ref[...],
                              preferred_element_type=jnp.float32)
    out_ref[...] = jnp.dot(hc, w_ref[...],
                           preferred_element_type=jnp.float32) + b_ref[...]


def _head_call(hcls8, mcls8, y, w, b):
    return pl.pallas_call(
        _head_body,
        out_shape=jax.ShapeDtypeStruct((8, 1024), jnp.float32),
    )(hcls8, mcls8, y, w, b)


def kernel(x, W_patch, b_patch, cls_token, pos_embed, ln1_s, ln1_b, Wqkv,
           bqkv, Wo, bo, ln2_s, ln2_b, gate_W, W1, b1, W2, b2, W_head,
           b_head):
    p = (x.reshape(BS, 3, 224 // PS, PS, 224 // PS, PS)
         .transpose(0, 2, 4, 1, 3, 5).reshape(BS * NPATCH, 3 * PS * PS))
    pos_t = jnp.tile(pos_embed[0, 1:, :], (BS, 1))
    t = _patch_call(p, W_patch, b_patch.reshape(1, DM), pos_t)
    cls = cls_token[0] + pos_embed[0, 0:1, :]
    h = jnp.concatenate(
        [jnp.broadcast_to(cls[None], (BS, 1, DM)), t.reshape(BS, NPATCH, DM)],
        axis=1)
    h = jnp.pad(h, ((0, 0), (0, SEQP - SEQ), (0, 0)))

    gwp = jnp.pad(gate_W, ((0, 0), (0, 0), (0, EPAD - NEXP)))
    l1s3 = ln1_s.reshape(NLAYER, 1, DM)
    l1b3 = ln1_b.reshape(NLAYER, 1, DM)
    bqkv3 = bqkv.reshape(NLAYER, 1, 3 * DM)
    bo3 = bo.reshape(NLAYER, 1, DM)
    l2s3 = ln2_s.reshape(NLAYER, 1, DM)
    l2b3 = ln2_b.reshape(NLAYER, 1, DM)
    b1r = b1.reshape(NLAYER, NEXP, 1, FF)
    b2r = b2.reshape(NLAYER, NEXP, 1, DM)

    y = jnp.zeros((NSORT, DM), jnp.float32)
    mt = jnp.zeros((NTOK, EPAD), jnp.float32)
    for i in range(NLAYER):
        li = jnp.full((1,), i, jnp.int32)
        h, z, perm8, mt, estart8, ecnt8 = _layer_call(
            li, h, y, mt, l1s3, l1b3, Wqkv, bqkv3, Wo, bo3, l2s3, l2b3, gwp)
        z_sorted = _sc_gather_kernel()(z, perm8[0])
        y = _ffn_call(li, estart8[0, :8], ecnt8[0, :8], z_sorted, W1, b1r,
                      W2, b2r)

    hcls = h.reshape(NTOK, DM)[::SEQP]
    hcls8 = jnp.pad(hcls, ((0, 8 - BS), (0, 0)))
    mcls8 = jnp.pad(mt[::SEQP], ((0, 8 - BS), (0, 0)))
    whp = jnp.pad(W_head, ((0, 0), (0, 1024 - NCLS)))
    bhp = jnp.pad(b_head, (0, 1024 - NCLS)).reshape(1, 1024)
    out = _head_call(hcls8, mcls8, y, whp, bhp)
    return out[:BS, :NCLS]
